# Initial kernel scaffold; baseline (speedup 1.0000x reference)
import jax
import jax.numpy as jnp
from jax.experimental import pallas as pl


def kernel(x, edge_index, edge_attr, batch, emb, rbf_W1, rbf_b1, rbf_W2, rbf_b2, cart_Wg, cart_bg, cart_Wm, cart_bm, mat_Wq, mat_Wk, mat_Wv, mat_We, mat_bW, mat_bb, cx_W, cx_b, cx_Wg, cx_bg, fc_W1, fc_b1, fc_W2, fc_b2):
    raise NotImplementedError("write your pallas kernel here")


# speedup vs baseline: 7.4646x; 7.4646x over previous
import functools

import jax
import jax.numpy as jnp
from jax import lax
from jax.experimental import pallas as pl
from jax.experimental.pallas import tpu as pltpu

N = 10000
E = 160000
H = 64
L = 3
HEADS = 4
DH = H // HEADS
G = 256
BINS = 128
FC = 128

BE = 1000
BN = 1000


def _sigmoid(x):
    return 1.0 / (1.0 + jnp.exp(-x))


def _mm(a, b):
    return jax.lax.dot_general(a, b, (((a.ndim - 1,), (0,)), ((), ())),
                               precision=jax.lax.Precision.HIGHEST)


def _eproj_body(ea_ref, W1_ref, b1_ref, W2P_ref, bP_ref, out_ref):
    ea = ea_ref[...]
    d2 = ea[:, 0:1] ** 2 + ea[:, 1:2] ** 2 + ea[:, 2:3] ** 2
    dist = jnp.sqrt(d2)
    step = 8.0 / (BINS - 1)
    gamma = 1.0 / (step * step)
    centers = lax.broadcasted_iota(
        jnp.int32, (1, BINS), 1).astype(jnp.float32) * step
    diff = dist - centers
    rbf = jnp.exp(-gamma * diff * diff)
    x = _mm(rbf, W1_ref[...]) + b1_ref[...]
    h1 = jnp.maximum(x, 0.0) + jnp.log1p(jnp.exp(-jnp.abs(x)))
    for i in range(L):
        out_ref[i] = _mm(h1, W2P_ref[i]) + bP_ref[i]


def _eproj(edge_attr, W1, b1, W2P, bP):
    grid = (E // BE,)
    return pl.pallas_call(
        _eproj_body,
        grid=grid,
        in_specs=[
            pl.BlockSpec((BE, 3), lambda j: (j, 0)),
            pl.BlockSpec((BINS, H), lambda j: (0, 0)),
            pl.BlockSpec((1, H), lambda j: (0, 0)),
            pl.BlockSpec((L, H, 3 * H), lambda j: (0, 0, 0)),
            pl.BlockSpec((L, 1, 3 * H), lambda j: (0, 0, 0)),
        ],
        out_specs=pl.BlockSpec((L, BE, 3 * H), lambda j: (0, j, 0)),
        out_shape=jax.ShapeDtypeStruct((L, E, 3 * H), jnp.float32),
    )(edge_attr, W1, b1, W2P, bP)


def _emb_body(x_ref, emb_ref, out_ref):
    xi = x_ref[...]
    iota = lax.broadcasted_iota(jnp.int32, (BN, 128), 1)
    oh = (iota == xi).astype(jnp.float32)
    out_ref[...] = _mm(oh, emb_ref[...])


def _embed(x2d, emb_pad):
    return pl.pallas_call(
        _emb_body,
        grid=(N // BN,),
        in_specs=[
            pl.BlockSpec((BN, 1), lambda j: (j, 0)),
            pl.BlockSpec((128, H), lambda j: (0, 0)),
        ],
        out_specs=pl.BlockSpec((BN, H), lambda j: (j, 0)),
        out_shape=jax.ShapeDtypeStruct((N, H), jnp.float32),
    )(x2d, emb_pad)


def _proj_body(xx_ref, Wd_ref, Ws_ref, td_ref, ts_ref):
    xx = xx_ref[...]
    td_ref[...] = _mm(xx, Wd_ref[...])
    ts_ref[...] = _mm(xx, Ws_ref[...])


def _proj(xx, Wd, Ws):
    return pl.pallas_call(
        _proj_body,
        grid=(N // BN,),
        in_specs=[
            pl.BlockSpec((BN, 2 * H), lambda j: (j, 0)),
            pl.BlockSpec((2 * H, 3 * H), lambda j: (0, 0)),
            pl.BlockSpec((2 * H, 4 * H), lambda j: (0, 0)),
        ],
        out_specs=[
            pl.BlockSpec((BN, 3 * H), lambda j: (j, 0)),
            pl.BlockSpec((BN, 4 * H), lambda j: (j, 0)),
        ],
        out_shape=[
            jax.ShapeDtypeStruct((N, 3 * H), jnp.float32),
            jax.ShapeDtypeStruct((N, 4 * H), jnp.float32),
        ],
    )(xx, Wd, Ws)


def _update_body(p0_ref, p1_ref, xc_ref, xm_ref, Wb_ref, bb_ref,
                 Wg_ref, bg_ref, Wf_ref, bf_ref, nxc_ref, nxm_ref):
    p = p0_ref[...] + p1_ref[...]
    agg = p[:, 0:H]
    num = p[:, H:2 * H]
    den = p[:, 2 * H:3 * H]
    xc = xc_ref[...]
    xm = xm_ref[...]
    out = num / (den + 1e-9)
    x_cart = xc + agg * _sigmoid(agg)
    beta = _sigmoid(_mm(xm, Wb_ref[0:H, :]) + _mm(out, Wb_ref[H:2 * H, :]) + bb_ref[...])
    x_mat = beta * xm + (1.0 - beta) * out
    g = _sigmoid(_mm(x_cart, Wg_ref[0:H, :]) + _mm(x_mat, Wg_ref[H:2 * H, :]) + bg_ref[...])
    fused = _mm(x_cart, Wf_ref[0:H, :]) + _mm(x_mat, Wf_ref[H:2 * H, :]) + bf_ref[...]
    x_out = g * fused + (1.0 - g) * (x_cart + x_mat) * 0.5
    nxc_ref[...] = x_out + x_cart
    nxm_ref[...] = x_out + x_mat


def _update(p0, p1, xc, xm, Wb, bb, Wg, bg, Wf, bf):
    wspec = pl.BlockSpec((2 * H, H), lambda j: (0, 0))
    bspec = pl.BlockSpec((1, H), lambda j: (0, 0))
    nspec = pl.BlockSpec((BN, H), lambda j: (j, 0))
    return pl.pallas_call(
        _update_body,
        grid=(N // BN,),
        in_specs=[
            pl.BlockSpec((BN, 3 * H), lambda j: (j, 0)),
            pl.BlockSpec((BN, 3 * H), lambda j: (j, 0)),
            nspec, nspec, wspec, bspec, wspec, bspec, wspec, bspec,
        ],
        out_specs=[nspec, nspec],
        out_shape=[
            jax.ShapeDtypeStruct((N, H), jnp.float32),
            jax.ShapeDtypeStruct((N, H), jnp.float32),
        ],
    )(p0, p1, xc, xm, Wb, bb, Wg, bg, Wf, bf)


def _pool_body(b_ref, xc_ref, W1_ref, b1_ref, W2_ref, b2_ref, out_ref,
               feats_acc, cnt_acc):
    j = pl.program_id(0)

    @pl.when(j == 0)
    def _init():
        feats_acc[...] = jnp.zeros_like(feats_acc)
        cnt_acc[...] = jnp.zeros_like(cnt_acc)

    b = b_ref[...]
    iota = lax.broadcasted_iota(jnp.int32, (BN, G), 1)
    oh = (iota == b).astype(jnp.float32)
    xc = xc_ref[...]
    feats_acc[...] += lax.dot_general(oh, xc, (((0,), (0,)), ((), ())),
                                      precision=lax.Precision.HIGHEST)
    cnt_acc[...] += lax.dot_general(
        oh, jnp.ones((BN, 1), jnp.float32), (((0,), (0,)), ((), ())),
        precision=lax.Precision.HIGHEST)

    @pl.when(j == N // BN - 1)
    def _fin():
        feats = feats_acc[...] / jnp.maximum(cnt_acc[...], 1.0)
        hid = _mm(feats, W1_ref[...]) + b1_ref[...]
        hid = hid * _sigmoid(hid)
        out_ref[...] = _mm(hid, W2_ref[...]) + b2_ref[...]


def _pool_fc(batch2d, xc, W1, b1, W2, b2):
    return pl.pallas_call(
        _pool_body,
        grid=(N // BN,),
        in_specs=[
            pl.BlockSpec((BN, 1), lambda j: (j, 0)),
            pl.BlockSpec((BN, H), lambda j: (j, 0)),
            pl.BlockSpec((H, FC), lambda j: (0, 0)),
            pl.BlockSpec((1, FC), lambda j: (0, 0)),
            pl.BlockSpec((FC, 1), lambda j: (0, 0)),
            pl.BlockSpec((1, 1), lambda j: (0, 0)),
        ],
        out_specs=pl.BlockSpec((G, 1), lambda j: (0, 0)),
        out_shape=jax.ShapeDtypeStruct((G, 1), jnp.float32),
        scratch_shapes=[
            pltpu.VMEM((G, H), jnp.float32),
            pltpu.VMEM((G, 1), jnp.float32),
        ],
    )(batch2d, xc, W1, b1, W2, b2)


def _edge_phase_jnp(Tdst, Tsrc, ep, src, dst):
    gd = jnp.take(Tdst, dst, axis=0)
    gs = jnp.take(Tsrc, src, axis=0)
    gate = _sigmoid(gd[:, 0:H] + gs[:, 0:H] + ep[:, 0:H])
    msg = (gd[:, H:2 * H] + gs[:, H:2 * H] + ep[:, H:2 * H]) * gate
    q = gd[:, 2 * H:3 * H].reshape(E, HEADS, DH)
    k = gs[:, 2 * H:3 * H].reshape(E, HEADS, DH)
    v = gs[:, 3 * H:4 * H].reshape(E, HEADS, DH)
    ee = ep[:, 2 * H:3 * H].reshape(E, HEADS, DH)
    logits = jnp.sum(q * (k + ee), axis=-1) / jnp.sqrt(float(DH))
    ex = jnp.exp(logits)
    numc = (ex[:, :, None] * (v + ee)).reshape(E, H)
    exb = jnp.repeat(ex, DH, axis=1)
    rows = jnp.concatenate([msg, numc, exb], axis=1)
    p = jax.ops.segment_sum(rows, dst, num_segments=N)
    return p, jnp.zeros_like(p)


def _fold_weights(rbf_W2, rbf_b2, cart_Wg, cart_bg, cart_Wm, cart_bm,
                  mat_Wq, mat_Wk, mat_Wv, mat_We, mat_bW, mat_bb,
                  cx_W, cx_b, cx_Wg, cx_bg):
    W2P, bP, Wd, Ws, Wb, Wgl, Wfl = [], [], [], [], [], [], []
    for i in range(L):
        P = jnp.concatenate(
            [cart_Wg[i][2 * H:3 * H], cart_Wm[i][2 * H:3 * H], mat_We[i]], axis=1)
        W2P.append(_mm(rbf_W2, P))
        bP.append((_mm(rbf_b2, P) + jnp.concatenate(
            [cart_bg[i], cart_bm[i], jnp.zeros((H,), jnp.float32)]))[None, :])
        zd = jnp.zeros((H, H), jnp.float32)
        Wd.append(jnp.block([
            [cart_Wg[i][0:H], cart_Wm[i][0:H], zd],
            [zd, zd, mat_Wq[i]],
        ]))
        Ws.append(jnp.block([
            [cart_Wg[i][H:2 * H], cart_Wm[i][H:2 * H], zd, zd],
            [zd, zd, mat_Wk[i], mat_Wv[i]],
        ]))
        Wb.append(jnp.concatenate(
            [mat_bW[i][0:H] + mat_bW[i][2 * H:3 * H],
             mat_bW[i][H:2 * H] - mat_bW[i][2 * H:3 * H]], axis=0))
        Wgl.append(cx_Wg[i])
        Wfl.append(cx_W[i])
    return (jnp.stack(W2P), jnp.stack(bP), Wd, Ws, Wb, Wgl, Wfl)


def kernel(x, edge_index, edge_attr, batch, emb, rbf_W1, rbf_b1, rbf_W2,
           rbf_b2, cart_Wg, cart_bg, cart_Wm, cart_bm, mat_Wq, mat_Wk,
           mat_Wv, mat_We, mat_bW, mat_bb, cx_W, cx_b, cx_Wg, cx_bg,
           fc_W1, fc_b1, fc_W2, fc_b2):
    src = edge_index[0]
    dst = edge_index[1]

    W2P, bP, Wd, Ws, Wb, Wgl, Wfl = _fold_weights(
        rbf_W2, rbf_b2, cart_Wg, cart_bg, cart_Wm, cart_bm,
        mat_Wq, mat_Wk, mat_Wv, mat_We, mat_bW, mat_bb,
        cx_W, cx_b, cx_Wg, cx_bg)

    emb_pad = jnp.zeros((128, H), jnp.float32).at[0:119].set(emb)
    x2d = x.astype(jnp.int32).reshape(N, 1)
    batch2d = batch.astype(jnp.int32).reshape(N, 1)

    eproj = _eproj(edge_attr, rbf_W1, rbf_b1.reshape(1, H), W2P, bP)
    h = _embed(x2d, emb_pad)

    x_c = h
    x_m = h
    for i in range(L):
        xx = jnp.concatenate([x_c, x_m], axis=1)
        Tdst, Tsrc = _proj(xx, Wd[i], Ws[i])
        p0, p1 = _edge_phase_jnp(Tdst, Tsrc, eproj[i], src, dst)
        x_c, x_m = _update(p0, p1, x_c, x_m,
                           Wb[i], mat_bb[i].reshape(1, H),
                           Wgl[i], cx_bg[i].reshape(1, H),
                           Wfl[i], cx_b[i].reshape(1, H))

    out = _pool_fc(batch2d, x_c, fc_W1, fc_b1.reshape(1, FC),
                   fc_W2, fc_b2.reshape(1, 1))
    return out.reshape(G)
